# baseline (speedup 1.0000x reference)
import functools

import jax
import jax.numpy as jnp
from jax import lax
from jax.experimental import pallas as pl
from jax.experimental.pallas import tpu as pltpu
from jax.experimental.pallas import tpu_sc as plsc

N = 2048
H = 1024
E = 8
DFF = 2730
T = 256
NT = 23
P = NT * T
NW = 32
TPW = N // NW
CHUNK = 32
DH = 1408
MAX_TILES_PER_EXPERT = 16


def _router_body(x_ref, rw_ref, rb_ref, logits_ref, idx_ref, w_ref):
    x = x_ref[...]
    logits = jnp.dot(x, rw_ref[...], preferred_element_type=jnp.float32)
    logits = logits + rb_ref[...]
    logits_ref[...] = logits
    m = jnp.max(logits, axis=1, keepdims=True)
    p = jnp.exp(logits - m)
    probs = p / jnp.sum(p, axis=1, keepdims=True)
    eio = lax.broadcasted_iota(jnp.int32, (N, E), 1)
    p1 = jnp.max(probs, axis=1, keepdims=True)
    i1 = jnp.min(jnp.where(probs == p1, eio, E), axis=1, keepdims=True)
    masked = jnp.where(eio == i1, -1.0, probs)
    p2 = jnp.max(masked, axis=1, keepdims=True)
    i2 = jnp.min(jnp.where(masked == p2, eio, E), axis=1, keepdims=True)
    s = p1 + p2
    w_ref[...] = jnp.concatenate([p1 / s, p2 / s], axis=1)
    idx_ref[...] = jnp.concatenate([i1, i2], axis=1)


def _router(x2d, router_w, router_b):
    return pl.pallas_call(
        _router_body,
        out_shape=(
            jax.ShapeDtypeStruct((N, E), jnp.float32),
            jax.ShapeDtypeStruct((N, 2), jnp.int32),
            jax.ShapeDtypeStruct((N, 2), jnp.float32),
        ),
    )(x2d, router_w, router_b.reshape(1, E))


HH = H // 2


def _mlp_body(te_ref, xg_ref, wg_a, wg_b, bg_ref, wu_a, wu_b, bu_ref,
              wd_a, wd_b, bd_ref, y_ref):
    xb = xg_ref[...].astype(jnp.bfloat16)
    h1 = (jnp.dot(xb[:, :HH], wg_a[0].astype(jnp.bfloat16),
                  preferred_element_type=jnp.float32)
          + jnp.dot(xb[:, HH:], wg_b[0].astype(jnp.bfloat16),
                    preferred_element_type=jnp.float32)) + bg_ref[0]
    h1b = h1.astype(jnp.bfloat16)
    h2 = (jnp.dot(h1b[:, :HH], wu_a[0].astype(jnp.bfloat16),
                  preferred_element_type=jnp.float32)
          + jnp.dot(h1b[:, HH:], wu_b[0].astype(jnp.bfloat16),
                    preferred_element_type=jnp.float32)) + bu_ref[0]
    h2 = h2 * jax.nn.sigmoid(h2)
    h2b = h2.astype(jnp.bfloat16)
    bd_v = bd_ref[0]
    y_ref[:, :HH] = jnp.dot(h2b, wd_a[0].astype(jnp.bfloat16),
                            preferred_element_type=jnp.float32) + bd_v[:, :HH]
    y_ref[:, HH:] = jnp.dot(h2b, wd_b[0].astype(jnp.bfloat16),
                            preferred_element_type=jnp.float32) + bd_v[:, HH:]


def _grouped_mlp(te, xg, Wg, bg, Wu, bu, Wd, bd):
    grid_spec = pltpu.PrefetchScalarGridSpec(
        num_scalar_prefetch=1,
        grid=(NT,),
        in_specs=[
            pl.BlockSpec((T, H), lambda i, te: (i, 0)),
            pl.BlockSpec((1, HH, H), lambda i, te: (te[i], 0, 0)),
            pl.BlockSpec((1, HH, H), lambda i, te: (te[i], 1, 0)),
            pl.BlockSpec((1, 1, H), lambda i, te: (te[i], 0, 0)),
            pl.BlockSpec((1, HH, DFF), lambda i, te: (te[i], 0, 0)),
            pl.BlockSpec((1, HH, DFF), lambda i, te: (te[i], 1, 0)),
            pl.BlockSpec((1, 1, DFF), lambda i, te: (te[i], 0, 0)),
            pl.BlockSpec((1, DFF, HH), lambda i, te: (te[i], 0, 0)),
            pl.BlockSpec((1, DFF, HH), lambda i, te: (te[i], 0, 1)),
            pl.BlockSpec((1, 1, H), lambda i, te: (te[i], 0, 0)),
        ],
        out_specs=pl.BlockSpec((T, H), lambda i, te: (i, 0)),
    )
    return pl.pallas_call(
        _mlp_body,
        grid_spec=grid_spec,
        out_shape=jax.ShapeDtypeStruct((P, H), jnp.float32),
        compiler_params=pltpu.CompilerParams(
            dimension_semantics=("arbitrary",),
            vmem_limit_bytes=63 * 1024 * 1024),
    )(te, xg, Wg, Wg, bg.reshape(E, 1, H), Wu, Wu, bu.reshape(E, 1, DFF),
      Wd, Wd, bd.reshape(E, 1, H))


def _dispatch(x2d, d0r, d1r):
    mesh = plsc.VectorSubcoreMesh(core_axis_name="c", subcore_axis_name="s")

    @functools.partial(
        pl.kernel,
        mesh=mesh,
        out_type=jax.ShapeDtypeStruct((P, H), jnp.float32),
        scratch_types=[
            pltpu.VMEM((TPW, H), jnp.float32),
            pltpu.VMEM((TPW,), jnp.int32),
            pltpu.VMEM((TPW,), jnp.int32),
            pltpu.SemaphoreType.DMA,
            pltpu.SemaphoreType.DMA,
        ],
    )
    def k(x_hbm, d0_hbm, d1_hbm, xg_hbm, rows_v, idx0_v, idx1_v, s0, s1):
        wid = lax.axis_index("c") * 16 + lax.axis_index("s")
        base = wid * TPW
        pltpu.sync_copy(x_hbm.at[pl.ds(base, TPW)], rows_v)
        pltpu.sync_copy(d0_hbm.at[wid], idx0_v)
        pltpu.sync_copy(d1_hbm.at[wid], idx1_v)
        c0 = pltpu.async_copy(rows_v, xg_hbm.at[idx0_v], s0)
        c1 = pltpu.async_copy(rows_v, xg_hbm.at[idx1_v], s1)
        c0.wait()
        c1.wait()

    return k(x2d, d0r, d1r)


def _combine(y, d0c, d1c, w0b, w1b):
    mesh = plsc.VectorSubcoreMesh(core_axis_name="c", subcore_axis_name="s")
    nch = TPW // CHUNK

    @functools.partial(
        pl.kernel,
        mesh=mesh,
        out_type=jax.ShapeDtypeStruct((N, H), jnp.float32),
        scratch_types=[
            pltpu.VMEM((CHUNK, H), jnp.float32),
            pltpu.VMEM((CHUNK, H), jnp.float32),
            pltpu.VMEM((CHUNK,), jnp.int32),
            pltpu.VMEM((CHUNK,), jnp.int32),
            pltpu.VMEM((TPW, 16), jnp.float32),
            pltpu.VMEM((TPW, 16), jnp.float32),
            pltpu.SemaphoreType.DMA,
            pltpu.SemaphoreType.DMA,
        ],
    )
    def k(y_hbm, d0_hbm, d1_hbm, w0_hbm, w1_hbm, out_hbm,
          r0, r1, idx0_v, idx1_v, w0_v, w1_v, s0, s1):
        wid = lax.axis_index("c") * 16 + lax.axis_index("s")
        base = wid * TPW
        pltpu.sync_copy(w0_hbm.at[wid], w0_v)
        pltpu.sync_copy(w1_hbm.at[wid], w1_v)
        for c in range(nch):
            pltpu.sync_copy(d0_hbm.at[wid, c], idx0_v)
            pltpu.sync_copy(d1_hbm.at[wid, c], idx1_v)
            c0 = pltpu.async_copy(y_hbm.at[idx0_v], r0, s0)
            c1 = pltpu.async_copy(y_hbm.at[idx1_v], r1, s1)
            c0.wait()
            c1.wait()

            def body(t, _):
                w0vec = w0_v[c * CHUNK + t]
                w1vec = w1_v[c * CHUNK + t]
                for kk in range(H // 16):
                    sl = pl.ds(kk * 16, 16)
                    r0[t, sl] = w0vec * r0[t, sl] + w1vec * r1[t, sl]
                return 0

            lax.fori_loop(0, CHUNK, body, 0)
            pltpu.sync_copy(r0, out_hbm.at[pl.ds(base + c * CHUNK, CHUNK)])

    return k(y, d0c, d1c, w0b, w1b)


def kernel(x, router_w, router_b, Wg, bg, Wu, bu, Wd, bd):
    Bx, Sx, Hx = x.shape
    x2d = x.reshape(N, H)
    logits, idxs, ws = _router(x2d, router_w, router_b)

    e_pair = idxs.reshape(-1)
    oh = (e_pair[:, None] == jnp.arange(E, dtype=jnp.int32)[None, :])
    oh = oh.astype(jnp.float32)
    incl = jnp.cumsum(oh, axis=0).astype(jnp.int32)
    oh = oh.astype(jnp.int32)
    rank = jnp.sum(incl * oh, axis=1) - 1
    counts = incl[-1]
    pc = ((counts + T - 1) // T) * T
    pend = jnp.cumsum(pc)
    pstart = pend - pc
    dest = (pstart[e_pair] + rank).astype(jnp.int32)

    tstart = jnp.arange(NT, dtype=jnp.int32) * T
    te = jnp.sum((tstart[:, None] >= pend[None, :]).astype(jnp.int32), axis=1)
    te = jnp.clip(te, 0, E - 1)
    used = pend[-1]
    te_last = te[jnp.maximum(used // T - 1, 0)]
    te = jnp.where(tstart >= used, te_last, te).astype(jnp.int32)

    dtok = dest.reshape(N, 2)
    d0r = dtok[:, 0].reshape(NW, TPW)
    d1r = dtok[:, 1].reshape(NW, TPW)
    xg = _dispatch(x2d, d0r, d1r)

    y = _grouped_mlp(te, xg, Wg, bg, Wu, bu, Wd, bd)

    nch = TPW // CHUNK
    d0c = dtok[:, 0].reshape(NW, nch, CHUNK)
    d1c = dtok[:, 1].reshape(NW, nch, CHUNK)
    w0b = jnp.broadcast_to(ws[:, 0:1], (N, 16)).reshape(NW, TPW, 16)
    w1b = jnp.broadcast_to(ws[:, 1:2], (N, 16)).reshape(NW, TPW, 16)
    out2d = _combine(y, d0c, d1c, w0b, w1b)

    return out2d.reshape(Bx, Sx, Hx), logits.reshape(Bx, Sx, E)


# speedup vs baseline: 1.0026x; 1.0026x over previous
import functools

import jax
import jax.numpy as jnp
from jax import lax
from jax.experimental import pallas as pl
from jax.experimental.pallas import tpu as pltpu
from jax.experimental.pallas import tpu_sc as plsc

N = 2048
H = 1024
E = 8
DFF = 2730
T = 256
NT = 23
P = NT * T
NW = 32
TPW = N // NW
CHUNK = 32
DH = 1408
MAX_TILES_PER_EXPERT = 16


def _router_body(x_ref, rw_ref, rb_ref, logits_ref, idx_ref, w_ref):
    x = x_ref[...]
    logits = jnp.dot(x, rw_ref[...], preferred_element_type=jnp.float32)
    logits = logits + rb_ref[...]
    logits_ref[...] = logits
    m = jnp.max(logits, axis=1, keepdims=True)
    p = jnp.exp(logits - m)
    probs = p / jnp.sum(p, axis=1, keepdims=True)
    eio = lax.broadcasted_iota(jnp.int32, (N, E), 1)
    p1 = jnp.max(probs, axis=1, keepdims=True)
    i1 = jnp.min(jnp.where(probs == p1, eio, E), axis=1, keepdims=True)
    masked = jnp.where(eio == i1, -1.0, probs)
    p2 = jnp.max(masked, axis=1, keepdims=True)
    i2 = jnp.min(jnp.where(masked == p2, eio, E), axis=1, keepdims=True)
    s = p1 + p2
    w_ref[...] = jnp.concatenate([p1 / s, p2 / s], axis=1)
    idx_ref[...] = jnp.concatenate([i1, i2], axis=1)


def _router(x2d, router_w, router_b):
    return pl.pallas_call(
        _router_body,
        out_shape=(
            jax.ShapeDtypeStruct((N, E), jnp.float32),
            jax.ShapeDtypeStruct((N, 2), jnp.int32),
            jax.ShapeDtypeStruct((N, 2), jnp.float32),
        ),
    )(x2d, router_w, router_b.reshape(1, E))


def _mlp_body(te_ref, xg_ref, wg_ref, bg_ref, wu_ref, bu_ref, wd_ref, bd_ref,
              y_ref):
    xb = xg_ref[...].astype(jnp.bfloat16)
    h1 = jnp.dot(xb, wg_ref[0].astype(jnp.bfloat16),
                 preferred_element_type=jnp.float32) + bg_ref[0]
    h2 = jnp.dot(h1.astype(jnp.bfloat16), wu_ref[0].astype(jnp.bfloat16),
                 preferred_element_type=jnp.float32) + bu_ref[0]
    h2 = h2 * jax.nn.sigmoid(h2)
    y_ref[...] = jnp.dot(h2.astype(jnp.bfloat16), wd_ref[0].astype(jnp.bfloat16),
                         preferred_element_type=jnp.float32) + bd_ref[0]


def _grouped_mlp(te, xg, Wg, bg, Wu, bu, Wd, bd):
    grid_spec = pltpu.PrefetchScalarGridSpec(
        num_scalar_prefetch=1,
        grid=(NT,),
        in_specs=[
            pl.BlockSpec((T, H), lambda i, te: (i, 0)),
            pl.BlockSpec((1, H, H), lambda i, te: (te[i], 0, 0)),
            pl.BlockSpec((1, 1, H), lambda i, te: (te[i], 0, 0)),
            pl.BlockSpec((1, H, DFF), lambda i, te: (te[i], 0, 0)),
            pl.BlockSpec((1, 1, DFF), lambda i, te: (te[i], 0, 0)),
            pl.BlockSpec((1, DFF, H), lambda i, te: (te[i], 0, 0)),
            pl.BlockSpec((1, 1, H), lambda i, te: (te[i], 0, 0)),
        ],
        out_specs=pl.BlockSpec((T, H), lambda i, te: (i, 0)),
    )
    return pl.pallas_call(
        _mlp_body,
        grid_spec=grid_spec,
        out_shape=jax.ShapeDtypeStruct((P, H), jnp.float32),
        compiler_params=pltpu.CompilerParams(
            dimension_semantics=("arbitrary",),
            vmem_limit_bytes=63 * 1024 * 1024),
    )(te, xg, Wg, bg.reshape(E, 1, H), Wu, bu.reshape(E, 1, DFF), Wd,
      bd.reshape(E, 1, H))


def _dispatch(x2d, d0r, d1r):
    mesh = plsc.VectorSubcoreMesh(core_axis_name="c", subcore_axis_name="s")

    @functools.partial(
        pl.kernel,
        mesh=mesh,
        out_type=jax.ShapeDtypeStruct((P, H), jnp.float32),
        scratch_types=[
            pltpu.VMEM((TPW, H), jnp.float32),
            pltpu.VMEM((TPW,), jnp.int32),
            pltpu.VMEM((TPW,), jnp.int32),
            pltpu.SemaphoreType.DMA,
            pltpu.SemaphoreType.DMA,
        ],
    )
    def k(x_hbm, d0_hbm, d1_hbm, xg_hbm, rows_v, idx0_v, idx1_v, s0, s1):
        wid = lax.axis_index("c") * 16 + lax.axis_index("s")
        base = wid * TPW
        pltpu.sync_copy(x_hbm.at[pl.ds(base, TPW)], rows_v)
        pltpu.sync_copy(d0_hbm.at[wid], idx0_v)
        pltpu.sync_copy(d1_hbm.at[wid], idx1_v)
        c0 = pltpu.async_copy(rows_v, xg_hbm.at[idx0_v], s0)
        c1 = pltpu.async_copy(rows_v, xg_hbm.at[idx1_v], s1)
        c0.wait()
        c1.wait()

    return k(x2d, d0r, d1r)


def _combine(y, d0c, d1c, w0b, w1b):
    mesh = plsc.VectorSubcoreMesh(core_axis_name="c", subcore_axis_name="s")
    nch = TPW // CHUNK

    @functools.partial(
        pl.kernel,
        mesh=mesh,
        out_type=jax.ShapeDtypeStruct((N, H), jnp.float32),
        scratch_types=[
            pltpu.VMEM((CHUNK, H), jnp.float32),
            pltpu.VMEM((CHUNK, H), jnp.float32),
            pltpu.VMEM((CHUNK,), jnp.int32),
            pltpu.VMEM((CHUNK,), jnp.int32),
            pltpu.VMEM((TPW, 16), jnp.float32),
            pltpu.VMEM((TPW, 16), jnp.float32),
            pltpu.SemaphoreType.DMA,
            pltpu.SemaphoreType.DMA,
        ],
    )
    def k(y_hbm, d0_hbm, d1_hbm, w0_hbm, w1_hbm, out_hbm,
          r0, r1, idx0_v, idx1_v, w0_v, w1_v, s0, s1):
        wid = lax.axis_index("c") * 16 + lax.axis_index("s")
        base = wid * TPW
        pltpu.sync_copy(w0_hbm.at[wid], w0_v)
        pltpu.sync_copy(w1_hbm.at[wid], w1_v)
        for c in range(nch):
            pltpu.sync_copy(d0_hbm.at[wid, c], idx0_v)
            pltpu.sync_copy(d1_hbm.at[wid, c], idx1_v)
            c0 = pltpu.async_copy(y_hbm.at[idx0_v], r0, s0)
            c1 = pltpu.async_copy(y_hbm.at[idx1_v], r1, s1)
            c0.wait()
            c1.wait()

            def body(t, _):
                w0vec = w0_v[c * CHUNK + t]
                w1vec = w1_v[c * CHUNK + t]
                for kk in range(H // 16):
                    sl = pl.ds(kk * 16, 16)
                    r0[t, sl] = w0vec * r0[t, sl] + w1vec * r1[t, sl]
                return 0

            lax.fori_loop(0, CHUNK, body, 0)
            pltpu.sync_copy(r0, out_hbm.at[pl.ds(base + c * CHUNK, CHUNK)])

    return k(y, d0c, d1c, w0b, w1b)


def kernel(x, router_w, router_b, Wg, bg, Wu, bu, Wd, bd):
    Bx, Sx, Hx = x.shape
    x2d = x.reshape(N, H)
    logits, idxs, ws = _router(x2d, router_w, router_b)

    e_pair = idxs.reshape(-1)
    oh = (e_pair[:, None] == jnp.arange(E, dtype=jnp.int32)[None, :])
    oh = oh.astype(jnp.int32)
    incl = jnp.cumsum(oh, axis=0)
    rank = jnp.sum(incl * oh, axis=1) - 1
    counts = incl[-1]
    pc = ((counts + T - 1) // T) * T
    pend = jnp.cumsum(pc)
    pstart = pend - pc
    dest = (pstart[e_pair] + rank).astype(jnp.int32)

    tstart = jnp.arange(NT, dtype=jnp.int32) * T
    te = jnp.sum((tstart[:, None] >= pend[None, :]).astype(jnp.int32), axis=1)
    te = jnp.clip(te, 0, E - 1)
    used = pend[-1]
    te_last = te[jnp.maximum(used // T - 1, 0)]
    te = jnp.where(tstart >= used, te_last, te).astype(jnp.int32)

    dtok = dest.reshape(N, 2)
    d0r = dtok[:, 0].reshape(NW, TPW)
    d1r = dtok[:, 1].reshape(NW, TPW)
    xg = _dispatch(x2d, d0r, d1r)

    y = _grouped_mlp(te, xg, Wg, bg, Wu, bu, Wd, bd)

    nch = TPW // CHUNK
    d0c = dtok[:, 0].reshape(NW, nch, CHUNK)
    d1c = dtok[:, 1].reshape(NW, nch, CHUNK)
    w0b = jnp.broadcast_to(ws[:, 0:1], (N, 16)).reshape(NW, TPW, 16)
    w1b = jnp.broadcast_to(ws[:, 1:2], (N, 16)).reshape(NW, TPW, 16)
    out2d = _combine(y, d0c, d1c, w0b, w1b)

    return out2d.reshape(Bx, Sx, Hx), logits.reshape(Bx, Sx, E)
